# baseline (speedup 1.0000x reference)
import jax
import jax.numpy as jnp
from jax import lax
from jax.experimental import pallas as pl
from jax.experimental.pallas import tpu as pltpu
from jax.experimental.pallas import tpu_sc as plsc

B = 32768
F = 512
GS = 64
ORDER = 3
MARGIN = 0.01
ALPHA = 0.02

NW = 32
RPW = F // NW
NB1 = 4096
SH1 = 20
NSLOT = 65
DC = 32
HC = NSLOT * DC
NH = 4
OUTW = 80
L = 16


def _transpose_body(x_ref, o_ref):
    xt = x_ref[...].T
    raw = lax.bitcast_convert_type(xt, jnp.int32)
    m = lax.shift_right_arithmetic(raw, 31)
    o_ref[...] = raw ^ (m | jnp.int32(-2147483648))


def _transpose(x):
    return pl.pallas_call(
        _transpose_body,
        grid=(F // 256, B // 1024),
        in_specs=[pl.BlockSpec((1024, 256), lambda i, j: (j, i))],
        out_specs=pl.BlockSpec((256, 1024), lambda i, j: (i, j)),
        out_shape=jax.ShapeDtypeStruct((F, B), jnp.int32),
    )(x)


def _sc_body(xt_hbm, lin_hbm, out_hbm, *refs):
    h1s = refs[0:NH]
    hcs = refs[NH:2 * NH]
    (key_v, gbuf_v, cntb_v, posb_v, cum1_v, act1_v, map1_v, cumc_v, actc_v,
     mapc_v, lin_v, val_v, out_v) = refs[2 * NH:]

    wid = lax.axis_index("s") * 2 + lax.axis_index("c")
    lane = lax.iota(jnp.int32, L)
    zeros_i = jnp.zeros((L,), jnp.int32)
    ones_i = jnp.ones((L,), jnp.int32)

    pltpu.sync_copy(lin_hbm, lin_v)

    @plsc.parallel_loop(0, NB1 // L, unroll=4)
    def zh1(i):
        for h in h1s:
            h[pl.ds(i * L, L)] = zeros_i
        act1_v[pl.ds(i * L, L)] = zeros_i

    @plsc.parallel_loop(0, HC // L, unroll=4)
    def zhc(i):
        for h in hcs:
            h[pl.ds(i * L, L)] = zeros_i
        actc_v[pl.ds(i * L, L)] = zeros_i

    def row_body(t, _):
        row = wid * RPW + t
        pltpu.sync_copy(xt_hbm.at[row], key_v)

        @plsc.parallel_loop(0, B // L, step=2 * NH)
        def pa(i):
            for u in range(2 * NH):
                o = (i + u) * L
                bn = lax.shift_right_logical(key_v[pl.ds(o, L)], SH1)
                plsc.addupdate_scatter(h1s[u % NH], [bn], ones_i)

        def ra(c2, tot):
            for u in range(2):
                o = (c2 * 2 + u) * L
                acc = h1s[0][pl.ds(o, L)]
                h1s[0][pl.ds(o, L)] = zeros_i
                for h in h1s[1:]:
                    acc = acc + h[pl.ds(o, L)]
                    h[pl.ds(o, L)] = zeros_i
                cs = plsc.cumsum(acc) + tot
                cum1_v[pl.ds(o, L)] = cs
                tot = tot + jnp.sum(acc)
            return tot
        lax.fori_loop(0, NB1 // (L * 2), ra, jnp.int32(0))

        qs, bins = [], []
        for g in range(GS // L):
            q = lin_v[pl.ds(g * L, L)]
            lo = zeros_i
            for step in (2048, 1024, 512, 256, 128, 64, 32, 16, 8, 4, 2, 1):
                cv = plsc.load_gather(cum1_v, [lo + (step - 1)])
                lo = lo + jnp.where(cv <= q, step, 0)
            cprev = plsc.load_gather(cum1_v, [jnp.maximum(lo - 1, 0)])
            q = q - jnp.where(lo > 0, cprev, 0)
            qs.append(q)
            bins.append(lo)

        for g in range(GS // L):
            plsc.addupdate_scatter(act1_v, [bins[g]], ones_i)

        def bm(c2, tot):
            for u in range(4):
                o = (c2 * 4 + u) * L
                v = act1_v[pl.ds(o, L)]
                act1_v[pl.ds(o, L)] = zeros_i
                cs = plsc.cumsum(v) + tot
                map1_v[pl.ds(o, L)] = jnp.where(v > 0, cs - 1, jnp.int32(64))
                tot = tot + jnp.sum(v)
            return tot
        lax.fori_loop(0, NB1 // (L * 4), bm, jnp.int32(0))

        slots = [plsc.load_gather(map1_v, [bins[g]]) for g in range(GS // L)]
        prefixes = [lax.shift_left(bins[g], SH1) for g in range(GS // L)]

        lane_is0 = lane == 0
        nvr = jnp.int32(B // L)
        for cj in range(4):
            sh = SH1 - 5 * cj

            if cj == 0:
                @plsc.parallel_loop(0, B // L, step=2 * NH)
                def pc0(i):
                    for u in range(2 * NH):
                        o = (i + u) * L
                        w = key_v[pl.ds(o, L)]
                        field = lax.shift_right_logical(w, SH1)
                        slot = plsc.load_gather(map1_v, [field])
                        digit = lax.shift_right_logical(w, SH1 - 5) & 31
                        plsc.addupdate_scatter(hcs[u % NH],
                                               [slot * DC + digit], ones_i)
                        marker = lax.shift_left(slot, 5) | digit
                        neww = lax.shift_left(marker, SH1 - 5) | \
                            (w & ((1 << (SH1 - 5)) - 1))
                        alive = slot < 64
                        plsc.store_compressed(key_v.at[pl.ds(o, L)], neww,
                                              mask=alive)
                        cnt = plsc.all_reduce_population_count(alive)
                        plsc.store_compressed(cntb_v.at[pl.ds(i + u, L)],
                                              cnt, mask=lane_is0)

                def pfx(c2, p):
                    for u in range(2):
                        o = (c2 * 2 + u) * L
                        v = cntb_v[pl.ds(o, L)]
                        cs = plsc.cumsum(v) + p
                        posb_v[pl.ds(o, L)] = cs - v
                        p = p + jnp.sum(v)
                    return p
                gtot = lax.fori_loop(0, B // (L * L * 2), pfx, jnp.int32(0))

                @plsc.parallel_loop(0, B // L, step=2)
                def rp(c):
                    for u in range(2):
                        ch = c + u
                        base = posb_v[pl.ds(ch, L)][0]
                        cnt = cntb_v[pl.ds(ch, L)][0]
                        v = key_v[pl.ds(ch * L, L)]
                        plsc.store_compressed(gbuf_v.at[pl.ds(base, L)], v,
                                              mask=lane < cnt)

                sent = jnp.full((L,), (64 << 5) << (SH1 - 5), jnp.int32)
                for k8 in range(2 * NH):
                    gbuf_v[pl.ds(gtot + k8 * L, L)] = sent
                nvr = lax.shift_left(
                    lax.shift_right_logical(gtot + (2 * NH * L - 1), 7), 3)
            else:
                @plsc.parallel_loop(0, nvr, step=2 * NH)
                def pc(i, sh=sh, cj=cj):
                    for u in range(2 * NH):
                        o = (i + u) * L
                        w = gbuf_v[pl.ds(o, L)]
                        field = lax.shift_right_logical(w, sh)
                        slot = plsc.load_gather(mapc_v, [field])
                        digit = lax.shift_right_logical(w, sh - 5) & 31 \
                            if sh > 5 else w & 31
                        plsc.addupdate_scatter(hcs[u % NH],
                                               [slot * DC + digit], ones_i)
                        if cj < 3:
                            marker = lax.shift_left(slot, 5) | digit
                            neww = lax.shift_left(marker, sh - 5) | \
                                (w & ((1 << (sh - 5)) - 1))
                            gbuf_v[pl.ds(o, L)] = neww

            def rc(s, cc):
                o0 = s * DC
                o1 = s * DC + L
                acc0 = hcs[0][pl.ds(o0, L)]
                acc1 = hcs[0][pl.ds(o1, L)]
                hcs[0][pl.ds(o0, L)] = zeros_i
                hcs[0][pl.ds(o1, L)] = zeros_i
                for h in hcs[1:]:
                    acc0 = acc0 + h[pl.ds(o0, L)]
                    acc1 = acc1 + h[pl.ds(o1, L)]
                    h[pl.ds(o0, L)] = zeros_i
                    h[pl.ds(o1, L)] = zeros_i
                cs0 = plsc.cumsum(acc0)
                cs1 = plsc.cumsum(acc1) + jnp.sum(acc0)
                cumc_v[pl.ds(o0, L)] = cs0
                cumc_v[pl.ds(o1, L)] = cs1
                return cc
            lax.fori_loop(0, 64, rc, 0)

            digits = []
            for g in range(GS // L):
                sbase = slots[g] * DC
                lo = zeros_i
                for step in (16, 8, 4, 2, 1):
                    cv = plsc.load_gather(cumc_v, [sbase + lo + (step - 1)])
                    lo = lo + jnp.where(cv <= qs[g], step, 0)
                cprev = plsc.load_gather(
                    cumc_v, [sbase + jnp.maximum(lo - 1, 0)])
                qs[g] = qs[g] - jnp.where(lo > 0, cprev, 0)
                prefixes[g] = prefixes[g] | lax.shift_left(lo, sh - 5)
                digits.append(lo)

            if cj < 3:
                for g in range(GS // L):
                    plsc.addupdate_scatter(
                        actc_v, [slots[g] * DC + digits[g]], ones_i)

                def bmc(c2, tot):
                    for u in range(2):
                        o = (c2 * 2 + u) * L
                        v = actc_v[pl.ds(o, L)]
                        actc_v[pl.ds(o, L)] = zeros_i
                        cs = plsc.cumsum(v) + tot
                        mapc_v[pl.ds(o, L)] = jnp.where(
                            v > 0, cs - 1, jnp.int32(64))
                        tot = tot + jnp.sum(v)
                    return tot
                lax.fori_loop(0, HC // (L * 2), bmc, jnp.int32(0))
                slots = [plsc.load_gather(mapc_v, [slots[g] * DC + digits[g]])
                         for g in range(GS // L)]

        for g in range(GS // L):
            p = prefixes[g]
            m = lax.shift_right_arithmetic(p, 31)
            bits = p ^ (jnp.int32(-2147483648) | ~m)
            val_v[pl.ds(g * L, L)] = lax.bitcast_convert_type(bits,
                                                              jnp.float32)

        t0 = val_v[pl.ds(0, L)]
        t3 = val_v[pl.ds(GS - L, L)]
        v0 = jnp.broadcast_to(jnp.sum(jnp.where(lane == 0, t0, 0.0)), (L,))
        v63 = jnp.broadcast_to(jnp.sum(jnp.where(lane == L - 1, t3, 0.0)),
                               (L,))
        ustep = (v63 - v0 + 2.0 * MARGIN) / (GS - 1)
        g0 = ALPHA * (v0 - MARGIN) + (1.0 - ALPHA) * v0
        g63 = ALPHA * ((GS - 1) * ustep + v0 - MARGIN) + (1.0 - ALPHA) * v63
        for c in range(OUTW // L):
            kk = lane + c * L
            i = jnp.clip(kk - ORDER, 0, GS - 1)
            a = plsc.load_gather(val_v, [i])
            i_f = i.astype(jnp.float32)
            k_f = kk.astype(jnp.float32)
            gb = ALPHA * (i_f * ustep + v0 - MARGIN) + (1.0 - ALPHA) * a
            pre = g0 - ustep * (float(ORDER) - k_f)
            post = g63 + ustep * (k_f - float(GS + ORDER - 1))
            res = jnp.where(kk < ORDER, pre,
                            jnp.where(kk > GS + ORDER - 1, post, gb))
            out_v[pl.ds(c * L, L)] = res
        pltpu.sync_copy(out_v, out_hbm.at[row])
        return 0

    lax.fori_loop(0, RPW, row_body, 0)


def _sc_select(xt, lin):
    mesh = plsc.VectorSubcoreMesh(core_axis_name="c", subcore_axis_name="s")
    kern = pl.kernel(
        _sc_body,
        out_type=jax.ShapeDtypeStruct((F, OUTW), jnp.float32),
        mesh=mesh,
        compiler_params=pltpu.CompilerParams(needs_layout_passes=False),
        scratch_types=(
            [pltpu.VMEM((NB1,), jnp.int32) for _ in range(NH)] +
            [pltpu.VMEM((HC,), jnp.int32) for _ in range(NH)] +
            [
                pltpu.VMEM((B,), jnp.int32),
                pltpu.VMEM((B + 2 * NH * L,), jnp.int32),
                pltpu.VMEM((B // L + L,), jnp.int32),
                pltpu.VMEM((B // L + L,), jnp.int32),
                pltpu.VMEM((NB1,), jnp.int32),
                pltpu.VMEM((NB1,), jnp.int32),
                pltpu.VMEM((NB1,), jnp.int32),
                pltpu.VMEM((HC,), jnp.int32),
                pltpu.VMEM((HC,), jnp.int32),
                pltpu.VMEM((HC,), jnp.int32),
                pltpu.VMEM((GS,), jnp.int32),
                pltpu.VMEM((GS,), jnp.float32),
                pltpu.VMEM((OUTW,), jnp.float32),
            ]
        ),
    )
    return kern(xt, lin)


def kernel(x):
    xt = _transpose(x)
    lin = jnp.linspace(0.0, float(B - 1), GS).astype(jnp.int32)
    out_t = _sc_select(xt, lin)
    return out_t[:, :GS + 2 * ORDER].T


# speedup vs baseline: 1.3236x; 1.3236x over previous
import jax
import jax.numpy as jnp
from jax import lax
from jax.experimental import pallas as pl
from jax.experimental.pallas import tpu as pltpu
from jax.experimental.pallas import tpu_sc as plsc

B = 32768
F = 512
GS = 64
ORDER = 3
MARGIN = 0.01
ALPHA = 0.02

NW = 32
RPW = F // NW
NB1 = 4096
SH1 = 20
NSLOT = 65
DC = 32
HC = NSLOT * DC
NH = 4
OUTW = 80
L = 16


def _transpose_body(x_ref, o_ref):
    xt = x_ref[...].T
    raw = lax.bitcast_convert_type(xt, jnp.int32)
    m = lax.shift_right_arithmetic(raw, 31)
    o_ref[...] = raw ^ (m | jnp.int32(-2147483648))


def _transpose(x):
    return pl.pallas_call(
        _transpose_body,
        grid=(F // 256, B // 1024),
        in_specs=[pl.BlockSpec((1024, 256), lambda i, j: (j, i))],
        out_specs=pl.BlockSpec((256, 1024), lambda i, j: (i, j)),
        out_shape=jax.ShapeDtypeStruct((F, B), jnp.int32),
    )(x)


def _sc_body(xt_hbm, lin_hbm, out_hbm, *refs):
    h1s = refs[0:NH]
    hcs = refs[NH:2 * NH]
    (key_v, cum1_v, act1_v, map1_v, cumc_v, actc_v, mapc_v,
     lin_v, val_v, out_v) = refs[2 * NH:]

    wid = lax.axis_index("s") * 2 + lax.axis_index("c")
    lane = lax.iota(jnp.int32, L)
    zeros_i = jnp.zeros((L,), jnp.int32)
    ones_i = jnp.ones((L,), jnp.int32)

    pltpu.sync_copy(lin_hbm, lin_v)

    @plsc.parallel_loop(0, NB1 // L, unroll=4)
    def zh1(i):
        for h in h1s:
            h[pl.ds(i * L, L)] = zeros_i
        act1_v[pl.ds(i * L, L)] = zeros_i

    @plsc.parallel_loop(0, HC // L, unroll=4)
    def zhc(i):
        for h in hcs:
            h[pl.ds(i * L, L)] = zeros_i
        actc_v[pl.ds(i * L, L)] = zeros_i

    def row_body(t, _):
        row = wid * RPW + t
        pltpu.sync_copy(xt_hbm.at[row], key_v)

        @plsc.parallel_loop(0, B // L, step=2 * NH)
        def pa(i):
            for u in range(2 * NH):
                o = (i + u) * L
                bn = lax.shift_right_logical(key_v[pl.ds(o, L)], SH1)
                plsc.addupdate_scatter(h1s[u % NH], [bn], ones_i)

        def ra(c2, tot):
            for u in range(2):
                o = (c2 * 2 + u) * L
                acc = h1s[0][pl.ds(o, L)]
                h1s[0][pl.ds(o, L)] = zeros_i
                for h in h1s[1:]:
                    acc = acc + h[pl.ds(o, L)]
                    h[pl.ds(o, L)] = zeros_i
                cs = plsc.cumsum(acc) + tot
                cum1_v[pl.ds(o, L)] = cs
                tot = tot + jnp.sum(acc)
            return tot
        lax.fori_loop(0, NB1 // (L * 2), ra, jnp.int32(0))

        qs, bins = [], []
        for g in range(GS // L):
            q = lin_v[pl.ds(g * L, L)]
            lo = zeros_i
            for step in (2048, 1024, 512, 256, 128, 64, 32, 16, 8, 4, 2, 1):
                cv = plsc.load_gather(cum1_v, [lo + (step - 1)])
                lo = lo + jnp.where(cv <= q, step, 0)
            cprev = plsc.load_gather(cum1_v, [jnp.maximum(lo - 1, 0)])
            q = q - jnp.where(lo > 0, cprev, 0)
            qs.append(q)
            bins.append(lo)

        for g in range(GS // L):
            plsc.addupdate_scatter(act1_v, [bins[g]], ones_i)

        def bm(c2, tot):
            for u in range(4):
                o = (c2 * 4 + u) * L
                v = act1_v[pl.ds(o, L)]
                act1_v[pl.ds(o, L)] = zeros_i
                cs = plsc.cumsum(v) + tot
                map1_v[pl.ds(o, L)] = jnp.where(v > 0, cs - 1, jnp.int32(64))
                tot = tot + jnp.sum(v)
            return tot
        lax.fori_loop(0, NB1 // (L * 4), bm, jnp.int32(0))

        slots = [plsc.load_gather(map1_v, [bins[g]]) for g in range(GS // L)]
        prefixes = [lax.shift_left(bins[g], SH1) for g in range(GS // L)]

        for cj in range(4):
            sh = SH1 - 5 * cj
            map_ref = map1_v if cj == 0 else mapc_v

            @plsc.parallel_loop(0, B // L, step=2 * NH)
            def pc(i, sh=sh, map_ref=map_ref, cj=cj):
                for u in range(2 * NH):
                    o = (i + u) * L
                    w = key_v[pl.ds(o, L)]
                    field = lax.shift_right_logical(w, sh)
                    slot = plsc.load_gather(map_ref, [field])
                    digit = lax.shift_right_logical(w, sh - 5) & 31 \
                        if sh > 5 else w & 31
                    plsc.addupdate_scatter(hcs[u % NH],
                                           [slot * DC + digit], ones_i)
                    if cj < 3:
                        marker = lax.shift_left(slot, 5) | digit
                        neww = lax.shift_left(marker, sh - 5) | \
                            (w & ((1 << (sh - 5)) - 1))
                        key_v[pl.ds(o, L)] = neww

            def rc(s, cc):
                o0 = s * DC
                o1 = s * DC + L
                acc0 = hcs[0][pl.ds(o0, L)]
                acc1 = hcs[0][pl.ds(o1, L)]
                hcs[0][pl.ds(o0, L)] = zeros_i
                hcs[0][pl.ds(o1, L)] = zeros_i
                for h in hcs[1:]:
                    acc0 = acc0 + h[pl.ds(o0, L)]
                    acc1 = acc1 + h[pl.ds(o1, L)]
                    h[pl.ds(o0, L)] = zeros_i
                    h[pl.ds(o1, L)] = zeros_i
                cs0 = plsc.cumsum(acc0)
                cs1 = plsc.cumsum(acc1) + jnp.sum(acc0)
                cumc_v[pl.ds(o0, L)] = cs0
                cumc_v[pl.ds(o1, L)] = cs1
                return cc
            lax.fori_loop(0, 64, rc, 0)

            digits = []
            for g in range(GS // L):
                sbase = slots[g] * DC
                lo = zeros_i
                for step in (16, 8, 4, 2, 1):
                    cv = plsc.load_gather(cumc_v, [sbase + lo + (step - 1)])
                    lo = lo + jnp.where(cv <= qs[g], step, 0)
                cprev = plsc.load_gather(
                    cumc_v, [sbase + jnp.maximum(lo - 1, 0)])
                qs[g] = qs[g] - jnp.where(lo > 0, cprev, 0)
                prefixes[g] = prefixes[g] | lax.shift_left(lo, sh - 5)
                digits.append(lo)

            if cj < 3:
                for g in range(GS // L):
                    plsc.addupdate_scatter(
                        actc_v, [slots[g] * DC + digits[g]], ones_i)

                def bmc(c2, tot):
                    for u in range(2):
                        o = (c2 * 2 + u) * L
                        v = actc_v[pl.ds(o, L)]
                        actc_v[pl.ds(o, L)] = zeros_i
                        cs = plsc.cumsum(v) + tot
                        mapc_v[pl.ds(o, L)] = jnp.where(
                            v > 0, cs - 1, jnp.int32(64))
                        tot = tot + jnp.sum(v)
                    return tot
                lax.fori_loop(0, HC // (L * 2), bmc, jnp.int32(0))
                slots = [plsc.load_gather(mapc_v, [slots[g] * DC + digits[g]])
                         for g in range(GS // L)]

        for g in range(GS // L):
            p = prefixes[g]
            m = lax.shift_right_arithmetic(p, 31)
            bits = p ^ (jnp.int32(-2147483648) | ~m)
            val_v[pl.ds(g * L, L)] = lax.bitcast_convert_type(bits,
                                                              jnp.float32)

        t0 = val_v[pl.ds(0, L)]
        t3 = val_v[pl.ds(GS - L, L)]
        v0 = jnp.broadcast_to(jnp.sum(jnp.where(lane == 0, t0, 0.0)), (L,))
        v63 = jnp.broadcast_to(jnp.sum(jnp.where(lane == L - 1, t3, 0.0)),
                               (L,))
        ustep = (v63 - v0 + 2.0 * MARGIN) / (GS - 1)
        g0 = ALPHA * (v0 - MARGIN) + (1.0 - ALPHA) * v0
        g63 = ALPHA * ((GS - 1) * ustep + v0 - MARGIN) + (1.0 - ALPHA) * v63
        for c in range(OUTW // L):
            kk = lane + c * L
            i = jnp.clip(kk - ORDER, 0, GS - 1)
            a = plsc.load_gather(val_v, [i])
            i_f = i.astype(jnp.float32)
            k_f = kk.astype(jnp.float32)
            gb = ALPHA * (i_f * ustep + v0 - MARGIN) + (1.0 - ALPHA) * a
            pre = g0 - ustep * (float(ORDER) - k_f)
            post = g63 + ustep * (k_f - float(GS + ORDER - 1))
            res = jnp.where(kk < ORDER, pre,
                            jnp.where(kk > GS + ORDER - 1, post, gb))
            out_v[pl.ds(c * L, L)] = res
        pltpu.sync_copy(out_v, out_hbm.at[row])
        return 0

    lax.fori_loop(0, RPW, row_body, 0)


def _sc_select(xt, lin):
    mesh = plsc.VectorSubcoreMesh(core_axis_name="c", subcore_axis_name="s")
    kern = pl.kernel(
        _sc_body,
        out_type=jax.ShapeDtypeStruct((F, OUTW), jnp.float32),
        mesh=mesh,
        compiler_params=pltpu.CompilerParams(needs_layout_passes=False),
        scratch_types=(
            [pltpu.VMEM((NB1,), jnp.int32) for _ in range(NH)] +
            [pltpu.VMEM((HC,), jnp.int32) for _ in range(NH)] +
            [
                pltpu.VMEM((B,), jnp.int32),
                pltpu.VMEM((NB1,), jnp.int32),
                pltpu.VMEM((NB1,), jnp.int32),
                pltpu.VMEM((NB1,), jnp.int32),
                pltpu.VMEM((HC,), jnp.int32),
                pltpu.VMEM((HC,), jnp.int32),
                pltpu.VMEM((HC,), jnp.int32),
                pltpu.VMEM((GS,), jnp.int32),
                pltpu.VMEM((GS,), jnp.float32),
                pltpu.VMEM((OUTW,), jnp.float32),
            ]
        ),
    )
    return kern(xt, lin)


def kernel(x):
    xt = _transpose(x)
    lin = jnp.linspace(0.0, float(B - 1), GS).astype(jnp.int32)
    out_t = _sc_select(xt, lin)
    return out_t[:, :GS + 2 * ORDER].T
